# baseline (speedup 1.0000x reference)
import functools

import jax
import jax.numpy as jnp
from jax import lax
from jax.experimental import pallas as pl
from jax.experimental.pallas import tpu as pltpu
from jax.experimental.pallas import tpu_sc as plsc

N_NODES = 10000
N_EDGES = 320000
D = 32
H = 8
ALPHA = 0.01

_ETYPES = [0, 1, 2, 3]
_EMAP = [0, 1, 2, 3]
_coef8 = [0.0] * (2 * len(_ETYPES))
for _i in range(len(_ETYPES) - 1):
    for _e in _ETYPES[_i:]:
        _coef8[_EMAP[_e]] += 1.0
_RVEC_COEF = tuple(_coef8[2 * _k] - _coef8[2 * _k + 1] for _k in range(len(_ETYPES)))

NCORES = 2
NTILES = 16
HPC = H // NCORES
E_PER_TILE = N_EDGES // NTILES
EBLK = 64
NBLK_TOT = N_EDGES // EBLK
NBLK_BASE = NBLK_TOT // NTILES
NBLK_REM = NBLK_TOT % NTILES
ROWS_PER_TILE = 640
N_PAD = NTILES * ROWS_PER_TILE

BT = 2000
_HP = lax.Precision.DEFAULT


def _edge_stage(f0, f1, f2, f3, ones_ref, attnt_ref, rvec_ref, hid_ref, p_ref):
    rv = rvec_ref[...]
    bias = jnp.zeros((1, D), jnp.float32)
    for k, ck in enumerate(_RVEC_COEF):
        if ck != 0.0:
            bias = bias + ck * rv[k:k + 1, :]
    acc = jnp.zeros((BT, D), jnp.float32)
    onesm = ones_ref[...]
    for f in (f0, f1, f2, f3):
        x = f[...]
        ss = jnp.dot(x * x, onesm, precision=_HP)
        acc = acc + x * lax.rsqrt(jnp.maximum(ss, 1e-24))
    hidden = acc * 0.25 + bias * 0.25
    hid_ref[...] = hidden
    a = jnp.dot(hidden, attnt_ref[...], precision=_HP)
    a = jnp.where(a >= 0, a, a * ALPHA)
    p = jnp.exp(a)
    p_ref[0] = jnp.concatenate(
        [p, jnp.zeros((BT, 16 - H), jnp.float32)], axis=1)
    p_ref[1] = jnp.concatenate(
        [p[:, HPC:], jnp.zeros((BT, 16 - HPC), jnp.float32)], axis=1)


_edge_call = pl.pallas_call(
    _edge_stage,
    grid=(N_EDGES // BT,),
    in_specs=[
        pl.BlockSpec((BT, D), lambda i: (i, 0)),
        pl.BlockSpec((BT, D), lambda i: (i, 0)),
        pl.BlockSpec((BT, D), lambda i: (i, 0)),
        pl.BlockSpec((BT, D), lambda i: (i, 0)),
        pl.BlockSpec((D, D), lambda i: (0, 0)),
        pl.BlockSpec((D, H), lambda i: (0, 0)),
        pl.BlockSpec((len(_ETYPES), D), lambda i: (0, 0)),
    ],
    out_specs=[
        pl.BlockSpec((BT, D), lambda i: (i, 0)),
        pl.BlockSpec((NCORES, BT, 16), lambda i: (0, i, 0)),
    ],
    out_shape=[
        jax.ShapeDtypeStruct((N_EDGES, D), jnp.float32),
        jax.ShapeDtypeStruct((NCORES, N_EDGES, 16), jnp.float32),
    ],
)


ASUM_ROWS8 = 10240 * H // 128
EBLK_A = 64
NBLK_A_TOT = N_EDGES // 2 // EBLK_A
NBLK_A_BASE = NBLK_A_TOT // NTILES
NBLK_A_REM = NBLK_A_TOT % NTILES
RBLK = EBLK // 4


def _num_stage(hid_hbm, p_hbm, dst_hbm, z2d_hbm, num_hbm,
               hid0, hid1, p0, p1, idx0, idx1, pay_v, acc, sem0, sem1):
    c = lax.axis_index("c")
    s = lax.axis_index("s")

    pltpu.sync_copy(z2d_hbm, acc.at[pl.ds(s * ROWS_PER_TILE, ROWS_PER_TILE)])
    plsc.subcore_barrier()

    bbase = s * NBLK_BASE + jnp.minimum(s, NBLK_REM)

    def issue(b, hv, pv, iv, sem):
        e0 = (bbase + b) * EBLK
        pltpu.async_copy(hid_hbm.at[pl.ds(e0, EBLK)], hv, sem)
        pltpu.async_copy(p_hbm.at[c].at[pl.ds(e0, EBLK)], pv, sem)
        pltpu.async_copy(dst_hbm.at[pl.ds(e0, EBLK)], iv, sem)

    def wait(b, hv, pv, iv, sem):
        e0 = (bbase + b) * EBLK
        pltpu.make_async_copy(hid_hbm.at[pl.ds(e0, EBLK)], hv, sem).wait()
        pltpu.make_async_copy(p_hbm.at[c].at[pl.ds(e0, EBLK)], pv, sem).wait()
        pltpu.make_async_copy(dst_hbm.at[pl.ds(e0, EBLK)], iv, sem).wait()

    def compute(hv, pvr, iv):
        for e in range(EBLK):
            pv = pvr[e]
            h0 = hv[e, 0:16]
            h1 = hv[e, 16:32]
            for j in range(HPC):
                ph = pv[j]
                pay_v[e, j * D:j * D + 16] = h0 * ph
                pay_v[e, j * D + 16:j * D + 32] = h1 * ph
        pltpu.sync_copy(pay_v, acc.at[iv], add=True)

    issue(0, hid0, p0, idx0, sem0)

    @pl.loop(0, NBLK_BASE - 2, step=2)
    def _(b):
        issue(b + 1, hid1, p1, idx1, sem1)
        wait(b, hid0, p0, idx0, sem0)
        compute(hid0, p0, idx0)
        issue(b + 2, hid0, p0, idx0, sem0)
        wait(b + 1, hid1, p1, idx1, sem1)
        compute(hid1, p1, idx1)

    issue(NBLK_BASE - 1, hid1, p1, idx1, sem1)
    wait(NBLK_BASE - 2, hid0, p0, idx0, sem0)
    compute(hid0, p0, idx0)
    wait(NBLK_BASE - 1, hid1, p1, idx1, sem1)
    compute(hid1, p1, idx1)

    @pl.when(s < NBLK_REM)
    def _():
        issue(NBLK_BASE, hid0, p0, idx0, sem0)
        wait(NBLK_BASE, hid0, p0, idx0, sem0)
        compute(hid0, p0, idx0)
    plsc.subcore_barrier()

    pltpu.sync_copy(
        acc.at[pl.ds(s * ROWS_PER_TILE, ROWS_PER_TILE)],
        num_hbm.at[c].at[pl.ds(s * ROWS_PER_TILE, ROWS_PER_TILE)],
    )


def _asum_stage(p_hbm, dst_hbm, z2d_hbm, arange_hbm, asum_hbm,
                p0, p1, idx0, idx1, aiv, asum_l, asum_s, sem0, sem1):
    c = lax.axis_index("c")
    s = lax.axis_index("s")
    lane = lax.iota(jnp.int32, 16)
    lmask = lane < H

    @pl.when(s == 0)
    def _():
        pltpu.sync_copy(z2d_hbm.at[pl.ds(0, ASUM_ROWS8)], asum_s)
    pltpu.sync_copy(z2d_hbm.at[pl.ds(0, ASUM_ROWS8)], asum_l)
    plsc.subcore_barrier()

    bbase = s * NBLK_A_BASE + jnp.minimum(s, NBLK_A_REM)

    def issue(b, pv, iv, sem):
        e0 = c * (N_EDGES // 2) + (bbase + b) * EBLK_A
        pltpu.async_copy(p_hbm.at[0].at[pl.ds(e0, EBLK_A)], pv, sem)
        pltpu.async_copy(dst_hbm.at[pl.ds(e0, EBLK_A)], iv, sem)

    def wait(b, pv, iv, sem):
        e0 = c * (N_EDGES // 2) + (bbase + b) * EBLK_A
        pltpu.make_async_copy(p_hbm.at[0].at[pl.ds(e0, EBLK_A)], pv, sem).wait()
        pltpu.make_async_copy(dst_hbm.at[pl.ds(e0, EBLK_A)], iv, sem).wait()

    def compute(pvr, iv):
        for g in range(EBLK_A // 16):
            dv = iv[pl.ds(g * 16, 16)]
            for k in range(16):
                e = g * 16 + k
                pv = pvr[e]
                aidx = dv[k] * H + lane
                plsc.addupdate_scatter(
                    asum_l, [aidx >> 7, aidx & 127], pv, mask=lmask)

    issue(0, p0, idx0, sem0)

    @pl.loop(0, NBLK_A_BASE - 2, step=2)
    def _(b):
        issue(b + 1, p1, idx1, sem1)
        wait(b, p0, idx0, sem0)
        compute(p0, idx0)
        issue(b + 2, p0, idx0, sem0)
        wait(b + 1, p1, idx1, sem1)
        compute(p1, idx1)

    issue(NBLK_A_BASE - 1, p1, idx1, sem1)
    wait(NBLK_A_BASE - 2, p0, idx0, sem0)
    compute(p0, idx0)
    wait(NBLK_A_BASE - 1, p1, idx1, sem1)
    compute(p1, idx1)

    @pl.when(s < NBLK_A_REM)
    def _():
        issue(NBLK_A_BASE, p0, idx0, sem0)
        wait(NBLK_A_BASE, p0, idx0, sem0)
        compute(p0, idx0)

    for m in range(ASUM_ROWS8 // 128):
        pltpu.sync_copy(arange_hbm.at[pl.ds(m * 128, 128)], aiv)
        pltpu.sync_copy(asum_l.at[pl.ds(m * 128, 128)], asum_s.at[aiv], add=True)
    plsc.subcore_barrier()

    @pl.when(s == 0)
    def _():
        pltpu.sync_copy(asum_s, asum_hbm.at[c])


@functools.cache
def _sc_calls():
    mesh = plsc.VectorSubcoreMesh(
        core_axis_name="c", subcore_axis_name="s",
        num_cores=NCORES, num_subcores=NTILES,
    )
    num_call = pl.kernel(
        _num_stage,
        out_type=jax.ShapeDtypeStruct((NCORES, N_PAD, HPC * D), jnp.float32),
        mesh=mesh,
        compiler_params=pltpu.CompilerParams(needs_layout_passes=False),
        scratch_types=[
            pltpu.VMEM((EBLK, D), jnp.float32),
            pltpu.VMEM((EBLK, D), jnp.float32),
            pltpu.VMEM((EBLK, 16), jnp.float32),
            pltpu.VMEM((EBLK, 16), jnp.float32),
            pltpu.VMEM((EBLK,), jnp.int32),
            pltpu.VMEM((EBLK,), jnp.int32),
            pltpu.VMEM((EBLK, HPC * D), jnp.float32),
            pltpu.VMEM_SHARED((N_PAD, HPC * D), jnp.float32),
            pltpu.SemaphoreType.DMA,
            pltpu.SemaphoreType.DMA,
        ],
    )
    asum_call = pl.kernel(
        _asum_stage,
        out_type=jax.ShapeDtypeStruct((NCORES, ASUM_ROWS8, 128), jnp.float32),
        mesh=mesh,
        compiler_params=pltpu.CompilerParams(needs_layout_passes=False),
        scratch_types=[
            pltpu.VMEM((EBLK_A, 16), jnp.float32),
            pltpu.VMEM((EBLK_A, 16), jnp.float32),
            pltpu.VMEM((EBLK_A,), jnp.int32),
            pltpu.VMEM((EBLK_A,), jnp.int32),
            pltpu.VMEM((128,), jnp.int32),
            pltpu.VMEM((ASUM_ROWS8, 128), jnp.float32),
            pltpu.VMEM_SHARED((ASUM_ROWS8, 128), jnp.float32),
            pltpu.SemaphoreType.DMA,
            pltpu.SemaphoreType.DMA,
        ],
    )
    return num_call, asum_call


BN = 2000


def _final_stage(num_ref, asum_ref, exp_ref, out_ref):
    den = asum_ref[0] + asum_ref[1]
    rec = jnp.where(den > 0, 1.0 / den, 0.0)
    f256 = jnp.dot(rec, exp_ref[...], precision=_HP)
    for cc in range(NCORES):
        out_ref[:, cc * 128:(cc + 1) * 128] = (
            num_ref[cc] * f256[:, cc * 128:(cc + 1) * 128])


_final_call = pl.pallas_call(
    _final_stage,
    grid=(N_NODES // BN,),
    in_specs=[
        pl.BlockSpec((NCORES, BN, HPC * D), lambda i: (0, i, 0)),
        pl.BlockSpec((NCORES, BN, H), lambda i: (0, i, 0)),
        pl.BlockSpec((H, H * D), lambda i: (0, 0)),
    ],
    out_specs=pl.BlockSpec((BN, H * D), lambda i: (i, 0)),
    out_shape=jax.ShapeDtypeStruct((N_NODES, H * D), jnp.float32),
)


def kernel(feat0, feat1, feat2, feat3, attn, r_vec, dst_idx):
    attn2 = attn.reshape(H, D)
    onesm = jnp.ones((D, D), jnp.float32)
    hidden, p_pad = _edge_call(feat0, feat1, feat2, feat3, onesm, attn2.T,
                               r_vec)
    z2d = jnp.zeros((ASUM_ROWS8, HPC * D), jnp.float32)
    arange = jnp.arange(ASUM_ROWS8, dtype=jnp.int32)
    num_call, asum_call = _sc_calls()
    dst32 = dst_idx.astype(jnp.int32)
    num = num_call(hidden, p_pad, dst32, z2d)
    asum = asum_call(p_pad, dst32, z2d, arange)
    jh = jnp.arange(H)[:, None]
    expander = (jnp.arange(H * D)[None, :] // D == jh).astype(jnp.float32)
    out = _final_call(num, asum.reshape(NCORES, ASUM_ROWS8 * 128 // H, H),
                      expander)
    return out.reshape(N_NODES, H, D)


# speedup vs baseline: 1.2640x; 1.2640x over previous
import functools

import jax
import jax.numpy as jnp
from jax import lax
from jax.experimental import pallas as pl
from jax.experimental.pallas import tpu as pltpu
from jax.experimental.pallas import tpu_sc as plsc

N_NODES = 10000
N_EDGES = 320000
D = 32
H = 8
ALPHA = 0.01

_ETYPES = [0, 1, 2, 3]
_EMAP = [0, 1, 2, 3]
_coef8 = [0.0] * (2 * len(_ETYPES))
for _i in range(len(_ETYPES) - 1):
    for _e in _ETYPES[_i:]:
        _coef8[_EMAP[_e]] += 1.0
_RVEC_COEF = tuple(_coef8[2 * _k] - _coef8[2 * _k + 1] for _k in range(len(_ETYPES)))

NCORES = 2
NTILES = 16
HPC = H // NCORES
E_PER_TILE = N_EDGES // NTILES
EBLK = 64
NBLK_TOT = N_EDGES // EBLK
NBLK_BASE = NBLK_TOT // NTILES
NBLK_REM = NBLK_TOT % NTILES
ROWS_PER_TILE = 640
N_PAD = NTILES * ROWS_PER_TILE

BT4 = 2000
_HP = lax.Precision.DEFAULT


def _edge_stage(f0, f1, f2, f3, seg_ref, w_ref, s1_ref, s2_ref, rvec_ref,
                hid_ref, p_ref):
    rv = rvec_ref[...]
    bias = jnp.zeros((1, D), jnp.float32)
    for k, ck in enumerate(_RVEC_COEF):
        if ck != 0.0:
            bias = bias + ck * rv[k:k + 1, :]
    bias128 = jnp.concatenate([bias] * 4, axis=1)
    acc = jnp.zeros((BT4, 128), jnp.float32)
    seg = seg_ref[...]
    for f in (f0, f1, f2, f3):
        x = f[...]
        ss = jnp.dot(x * x, seg, precision=_HP)
        acc = acc + x * lax.rsqrt(jnp.maximum(ss, 1e-24))
    hidden = acc * 0.25 + bias128 * 0.25
    hid_ref[...] = hidden
    a = jnp.dot(hidden, w_ref[...], precision=_HP)
    a = jnp.where(a >= 0, a, a * ALPHA)
    p = jnp.exp(a)
    p_ref[0] = jnp.dot(p, s1_ref[...], precision=_HP)
    p_ref[1] = jnp.dot(p, s2_ref[...], precision=_HP)


_edge_call = pl.pallas_call(
    _edge_stage,
    grid=(N_EDGES // 4 // BT4,),
    in_specs=[
        pl.BlockSpec((BT4, 128), lambda i: (i, 0)),
        pl.BlockSpec((BT4, 128), lambda i: (i, 0)),
        pl.BlockSpec((BT4, 128), lambda i: (i, 0)),
        pl.BlockSpec((BT4, 128), lambda i: (i, 0)),
        pl.BlockSpec((128, 128), lambda i: (0, 0)),
        pl.BlockSpec((128, 32), lambda i: (0, 0)),
        pl.BlockSpec((32, 64), lambda i: (0, 0)),
        pl.BlockSpec((32, 64), lambda i: (0, 0)),
        pl.BlockSpec((len(_ETYPES), D), lambda i: (0, 0)),
    ],
    out_specs=[
        pl.BlockSpec((BT4, 128), lambda i: (i, 0)),
        pl.BlockSpec((NCORES, BT4, 64), lambda i: (0, i, 0)),
    ],
    out_shape=[
        jax.ShapeDtypeStruct((N_EDGES // 4, 128), jnp.float32),
        jax.ShapeDtypeStruct((NCORES, N_EDGES // 4, 64), jnp.float32),
    ],
)


def _edge_consts(attn2):
    g32 = jnp.arange(128, dtype=jnp.int32) // 32
    seg = (g32[:, None] == g32[None, :]).astype(jnp.float32)
    wt = attn2.T
    z = jnp.zeros((32, 8), jnp.float32)
    w128 = jnp.concatenate(
        [jnp.concatenate([wt if gi == gj else z for gj in range(4)], axis=1)
         for gi in range(4)], axis=0)
    r = jnp.arange(32)
    cA = (r // 8) * 16 + (r % 8)
    s1 = (cA[:, None] == jnp.arange(64)[None, :]).astype(jnp.float32)
    cB = jnp.where(r % 8 >= 4, (r // 8) * 16 + (r % 8) - 4, -1)
    s2 = (cB[:, None] == jnp.arange(64)[None, :]).astype(jnp.float32)
    return seg, w128, s1, s2


ASUM_ROWS8 = 10240 * H // 128
EBLK_A = 64
NBLK_A_TOT = N_EDGES // 2 // EBLK_A
NBLK_A_BASE = NBLK_A_TOT // NTILES
NBLK_A_REM = NBLK_A_TOT % NTILES
RBLK = EBLK // 4


def _num_stage(hid_hbm, p_hbm, dst_hbm, z2d_hbm, num_hbm,
               hid0, hid1, p0, p1, idx0, idx1, pay_v, acc, sem0, sem1):
    c = lax.axis_index("c")
    s = lax.axis_index("s")

    pltpu.sync_copy(z2d_hbm, acc.at[pl.ds(s * ROWS_PER_TILE, ROWS_PER_TILE)])
    plsc.subcore_barrier()

    bbase = s * NBLK_BASE + jnp.minimum(s, NBLK_REM)

    def issue(b, hv, pv, iv, sem):
        e0 = (bbase + b) * EBLK
        r0 = (bbase + b) * RBLK
        pltpu.async_copy(hid_hbm.at[pl.ds(r0, RBLK)], hv, sem)
        pltpu.async_copy(p_hbm.at[c].at[pl.ds(r0, RBLK)], pv, sem)
        pltpu.async_copy(dst_hbm.at[pl.ds(e0, EBLK)], iv, sem)

    def wait(b, hv, pv, iv, sem):
        e0 = (bbase + b) * EBLK
        r0 = (bbase + b) * RBLK
        pltpu.make_async_copy(hid_hbm.at[pl.ds(r0, RBLK)], hv, sem).wait()
        pltpu.make_async_copy(p_hbm.at[c].at[pl.ds(r0, RBLK)], pv, sem).wait()
        pltpu.make_async_copy(dst_hbm.at[pl.ds(e0, EBLK)], iv, sem).wait()

    def compute(hv, pvr, iv):
        for e in range(EBLK):
            hr, q = e >> 2, (e & 3)
            pv = pvr[hr, q * 16:q * 16 + 16]
            h0 = hv[hr, q * 32:q * 32 + 16]
            h1 = hv[hr, q * 32 + 16:q * 32 + 32]
            for j in range(HPC):
                ph = pv[j]
                pay_v[e, j * D:j * D + 16] = h0 * ph
                pay_v[e, j * D + 16:j * D + 32] = h1 * ph
        pltpu.sync_copy(pay_v, acc.at[iv], add=True)

    issue(0, hid0, p0, idx0, sem0)

    @pl.loop(0, NBLK_BASE - 2, step=2)
    def _(b):
        issue(b + 1, hid1, p1, idx1, sem1)
        wait(b, hid0, p0, idx0, sem0)
        compute(hid0, p0, idx0)
        issue(b + 2, hid0, p0, idx0, sem0)
        wait(b + 1, hid1, p1, idx1, sem1)
        compute(hid1, p1, idx1)

    issue(NBLK_BASE - 1, hid1, p1, idx1, sem1)
    wait(NBLK_BASE - 2, hid0, p0, idx0, sem0)
    compute(hid0, p0, idx0)
    wait(NBLK_BASE - 1, hid1, p1, idx1, sem1)
    compute(hid1, p1, idx1)

    @pl.when(s < NBLK_REM)
    def _():
        issue(NBLK_BASE, hid0, p0, idx0, sem0)
        wait(NBLK_BASE, hid0, p0, idx0, sem0)
        compute(hid0, p0, idx0)
    plsc.subcore_barrier()

    pltpu.sync_copy(
        acc.at[pl.ds(s * ROWS_PER_TILE, ROWS_PER_TILE)],
        num_hbm.at[c].at[pl.ds(s * ROWS_PER_TILE, ROWS_PER_TILE)],
    )


def _asum_stage(p_hbm, dst_hbm, z2d_hbm, arange_hbm, asum_hbm,
                p0, p1, idx0, idx1, aiv, asum_l, asum_s, sem0, sem1):
    c = lax.axis_index("c")
    s = lax.axis_index("s")
    lane = lax.iota(jnp.int32, 16)
    lmask = lane < H

    @pl.when(s == 0)
    def _():
        pltpu.sync_copy(z2d_hbm.at[pl.ds(0, ASUM_ROWS8)], asum_s)
    pltpu.sync_copy(z2d_hbm.at[pl.ds(0, ASUM_ROWS8)], asum_l)
    plsc.subcore_barrier()

    bbase = s * NBLK_A_BASE + jnp.minimum(s, NBLK_A_REM)

    def issue(b, pv, iv, sem):
        e0 = c * (N_EDGES // 2) + (bbase + b) * EBLK_A
        r0 = c * (N_EDGES // 8) + (bbase + b) * (EBLK_A // 4)
        pltpu.async_copy(p_hbm.at[0].at[pl.ds(r0, EBLK_A // 4)], pv, sem)
        pltpu.async_copy(dst_hbm.at[pl.ds(e0, EBLK_A)], iv, sem)

    def wait(b, pv, iv, sem):
        e0 = c * (N_EDGES // 2) + (bbase + b) * EBLK_A
        r0 = c * (N_EDGES // 8) + (bbase + b) * (EBLK_A // 4)
        pltpu.make_async_copy(p_hbm.at[0].at[pl.ds(r0, EBLK_A // 4)], pv, sem).wait()
        pltpu.make_async_copy(dst_hbm.at[pl.ds(e0, EBLK_A)], iv, sem).wait()

    def compute(pvr, iv):
        for g in range(EBLK_A // 16):
            dv = iv[pl.ds(g * 16, 16)]
            for k in range(16):
                e = g * 16 + k
                pv = pvr[e >> 2, (e & 3) * 16:(e & 3) * 16 + 16]
                aidx = dv[k] * H + lane
                plsc.addupdate_scatter(
                    asum_l, [aidx >> 7, aidx & 127], pv, mask=lmask)

    issue(0, p0, idx0, sem0)

    @pl.loop(0, NBLK_A_BASE - 2, step=2)
    def _(b):
        issue(b + 1, p1, idx1, sem1)
        wait(b, p0, idx0, sem0)
        compute(p0, idx0)
        issue(b + 2, p0, idx0, sem0)
        wait(b + 1, p1, idx1, sem1)
        compute(p1, idx1)

    issue(NBLK_A_BASE - 1, p1, idx1, sem1)
    wait(NBLK_A_BASE - 2, p0, idx0, sem0)
    compute(p0, idx0)
    wait(NBLK_A_BASE - 1, p1, idx1, sem1)
    compute(p1, idx1)

    @pl.when(s < NBLK_A_REM)
    def _():
        issue(NBLK_A_BASE, p0, idx0, sem0)
        wait(NBLK_A_BASE, p0, idx0, sem0)
        compute(p0, idx0)

    for m in range(ASUM_ROWS8 // 128):
        pltpu.sync_copy(arange_hbm.at[pl.ds(m * 128, 128)], aiv)
        pltpu.sync_copy(asum_l.at[pl.ds(m * 128, 128)], asum_s.at[aiv], add=True)
    plsc.subcore_barrier()

    @pl.when(s == 0)
    def _():
        pltpu.sync_copy(asum_s, asum_hbm.at[c])


@functools.cache
def _sc_calls():
    mesh = plsc.VectorSubcoreMesh(
        core_axis_name="c", subcore_axis_name="s",
        num_cores=NCORES, num_subcores=NTILES,
    )
    num_call = pl.kernel(
        _num_stage,
        out_type=jax.ShapeDtypeStruct((NCORES, N_PAD, HPC * D), jnp.float32),
        mesh=mesh,
        compiler_params=pltpu.CompilerParams(needs_layout_passes=False),
        scratch_types=[
            pltpu.VMEM((RBLK, 128), jnp.float32),
            pltpu.VMEM((RBLK, 128), jnp.float32),
            pltpu.VMEM((RBLK, 64), jnp.float32),
            pltpu.VMEM((RBLK, 64), jnp.float32),
            pltpu.VMEM((EBLK,), jnp.int32),
            pltpu.VMEM((EBLK,), jnp.int32),
            pltpu.VMEM((EBLK, HPC * D), jnp.float32),
            pltpu.VMEM_SHARED((N_PAD, HPC * D), jnp.float32),
            pltpu.SemaphoreType.DMA,
            pltpu.SemaphoreType.DMA,
        ],
    )
    asum_call = pl.kernel(
        _asum_stage,
        out_type=jax.ShapeDtypeStruct((NCORES, ASUM_ROWS8, 128), jnp.float32),
        mesh=mesh,
        compiler_params=pltpu.CompilerParams(needs_layout_passes=False),
        scratch_types=[
            pltpu.VMEM((EBLK_A // 4, 64), jnp.float32),
            pltpu.VMEM((EBLK_A // 4, 64), jnp.float32),
            pltpu.VMEM((EBLK_A,), jnp.int32),
            pltpu.VMEM((EBLK_A,), jnp.int32),
            pltpu.VMEM((128,), jnp.int32),
            pltpu.VMEM((ASUM_ROWS8, 128), jnp.float32),
            pltpu.VMEM_SHARED((ASUM_ROWS8, 128), jnp.float32),
            pltpu.SemaphoreType.DMA,
            pltpu.SemaphoreType.DMA,
        ],
    )
    return num_call, asum_call


BN = 2000


def _final_stage(num_ref, asum_ref, exp_ref, out_ref):
    den = asum_ref[0] + asum_ref[1]
    rec = jnp.where(den > 0, 1.0 / den, 0.0)
    f256 = jnp.dot(rec, exp_ref[...], precision=_HP)
    for cc in range(NCORES):
        out_ref[:, cc * 128:(cc + 1) * 128] = (
            num_ref[cc] * f256[:, cc * 128:(cc + 1) * 128])


_final_call = pl.pallas_call(
    _final_stage,
    grid=(N_NODES // BN,),
    in_specs=[
        pl.BlockSpec((NCORES, BN, HPC * D), lambda i: (0, i, 0)),
        pl.BlockSpec((NCORES, BN, H), lambda i: (0, i, 0)),
        pl.BlockSpec((H, H * D), lambda i: (0, 0)),
    ],
    out_specs=pl.BlockSpec((BN, H * D), lambda i: (i, 0)),
    out_shape=jax.ShapeDtypeStruct((N_NODES, H * D), jnp.float32),
)


def kernel(feat0, feat1, feat2, feat3, attn, r_vec, dst_idx):
    attn2 = attn.reshape(H, D)
    seg, w128, s1, s2 = _edge_consts(attn2)
    f4 = [f.reshape(N_EDGES // 4, 128) for f in (feat0, feat1, feat2, feat3)]
    hidden4, p4 = _edge_call(*f4, seg, w128, s1, s2, r_vec)
    z2d = jnp.zeros((ASUM_ROWS8, HPC * D), jnp.float32)
    arange = jnp.arange(ASUM_ROWS8, dtype=jnp.int32)
    num_call, asum_call = _sc_calls()
    dst32 = dst_idx.astype(jnp.int32)
    num = num_call(hidden4, p4, dst32, z2d)
    asum = asum_call(p4, dst32, z2d, arange)
    jh = jnp.arange(H)[:, None]
    expander = (jnp.arange(H * D)[None, :] // D == jh).astype(jnp.float32)
    out = _final_call(num, asum.reshape(NCORES, ASUM_ROWS8 * 128 // H, H),
                      expander)
    return out.reshape(N_NODES, H, D)
